# baseline (speedup 1.0000x reference)
import functools

import jax
import jax.numpy as jnp
from jax import lax
from jax.experimental import pallas as pl
from jax.experimental.pallas import tpu as pltpu
from jax.experimental.pallas import tpu_sc as plsc

D = 128
NN = 10000
NNP = 10240
NRED = NNP // 16
NE = 320000
NB = 100000
NC, NS, L = 2, 16, 16
NW = NC * NS
BPW = 3200
BPAD = NW * BPW
CH = 64
NCH = BPW // CH
NCH2 = NCH // 2
EPT = NE // NS
RJ = D // L
MW = 16

_mesh = plsc.VectorSubcoreMesh(core_axis_name="c", subcore_axis_name="s")
_sc_params = pltpu.CompilerParams(needs_layout_passes=False)
_sc_params_untiled = pltpu.CompilerParams(needs_layout_passes=False,
                                          use_tc_tiling_on_sc=False)


@functools.partial(
    pl.kernel,
    out_type=(jax.ShapeDtypeStruct((NNP, D), jnp.float32),
              jax.ShapeDtypeStruct((NNP, D), jnp.float32)),
    mesh=_mesh,
    scratch_types=[
        pltpu.VMEM((EPT,), jnp.int32),
        pltpu.VMEM((NNP,), jnp.float32),
        pltpu.VMEM((NS, NRED), jnp.float32),
        pltpu.VMEM((NRED,), jnp.float32),
        pltpu.VMEM((128, D), jnp.float32),
        pltpu.VMEM_SHARED((NS, NNP), jnp.float32),
    ],
    compiler_params=_sc_params,
)
def _edge_mask_kernel(eidx_a, eidx_b, out_a, out_b, idx_v,
                      mask_v, red_v, res_v, bc_v, shared):
    cid = lax.axis_index("c")
    sid = lax.axis_index("s")
    zeros16 = jnp.zeros((L,), jnp.float32)
    ones16 = jnp.ones((L,), jnp.float32)

    def run(eidx_hbm, out_hbm):
        pltpu.sync_copy(eidx_hbm.at[pl.ds(sid * EPT, EPT)], idx_v)

        def zero_body(i, carry):
            for u in range(4):
                mask_v[pl.ds((4 * i + u) * L, L)] = zeros16
            return carry

        lax.fori_loop(0, NNP // (4 * L), zero_body, 0)

        def scat_body(i, carry):
            for u in range(4):
                idx = idx_v[pl.ds((4 * i + u) * L, L)]
                plsc.store_scatter(mask_v, [idx], ones16)
            return carry

        lax.fori_loop(0, EPT // (4 * L), scat_body, 0)

        pltpu.sync_copy(mask_v, shared.at[sid])
        plsc.subcore_barrier()
        for s in range(NS):
            pltpu.sync_copy(shared.at[s, pl.ds(sid * NRED, NRED)], red_v.at[s])

        def red_body(j, carry):
            m = red_v[0, pl.ds(j * L, L)]
            for s in range(1, NS):
                m = jnp.maximum(m, red_v[s, pl.ds(j * L, L)])
            res_v[pl.ds(j * L, L)] = m
            return carry

        lax.fori_loop(0, NRED // L, red_body, 0)

        for ch in range(NRED // 128):
            def grp_body(k, carry):
                mv = res_v[pl.ds(ch * 128 + k * L, L)]
                for j in range(L):
                    bc_v[k * L + j, pl.ds(0, L)] = jnp.broadcast_to(mv[j], (L,))
                return carry

            lax.fori_loop(0, 128 // L, grp_body, 0)
            pltpu.sync_copy(bc_v, out_hbm.at[pl.ds(sid * NRED + ch * 128, 128)])

    @pl.when(cid == 0)
    def _():
        run(eidx_a, out_a)

    @pl.when(cid == 1)
    def _():
        run(eidx_b, out_b)


@functools.partial(
    pl.kernel,
    out_type=jax.ShapeDtypeStruct((BPAD,), jnp.float32),
    mesh=_mesh,
    scratch_types=[
        pltpu.VMEM((NCH, CH), jnp.int32),
        pltpu.VMEM((NCH, CH), jnp.int32),
        pltpu.VMEM((CH, D // 2), jnp.int32),
        pltpu.VMEM((CH, D // 2), jnp.int32),
        pltpu.VMEM((CH, D // 2), jnp.int32),
        pltpu.VMEM((CH, D // 2), jnp.int32),
        pltpu.VMEM((BPW,), jnp.float32),
        pltpu.VMEM((D,), jnp.float32),
        pltpu.VMEM((L,), jnp.float32),
        pltpu.SemaphoreType.DMA,
        pltpu.SemaphoreType.DMA,
        pltpu.SemaphoreType.DMA,
        pltpu.SemaphoreType.DMA,
    ],
    compiler_params=_sc_params_untiled,
)
def _decoder_kernel(g_hbm, dz_hbm, rows_hbm, cols_hbm, w2_hbm, b2_hbm, out_hbm,
                    rowi_v, coli_v, gbuf0, dbuf0, gbuf1, dbuf1, outv, w2_v,
                    b2_v, sg0, sd0, sg1, sd1):
    wid = lax.axis_index("s") * NC + lax.axis_index("c")
    pltpu.sync_copy(rows_hbm.at[wid], rowi_v)
    pltpu.sync_copy(cols_hbm.at[wid], coli_v)
    pltpu.sync_copy(w2_hbm, w2_v)
    pltpu.sync_copy(b2_hbm, b2_v)
    b2vec = b2_v[...]
    w2r = [w2_v[pl.ds(L * j, L)] for j in range(RJ)]
    lane = lax.iota(jnp.int32, L)
    slots = ((gbuf0, dbuf0, sg0, sd0), (gbuf1, dbuf1, sg1, sd1))

    def issue(slot, c):
        gb, db, sg, sd = slot
        pltpu.async_copy(g_hbm.at[rowi_v.at[c]], gb, sg)
        pltpu.async_copy(dz_hbm.at[coli_v.at[c]], db, sd)

    def wait_slot(slot):
        gb, db, sg, sd = slot
        pltpu.make_async_copy(g_hbm.at[rowi_v.at[0]], gb, sg).wait()
        pltpu.make_async_copy(dz_hbm.at[coli_v.at[0]], db, sd).wait()

    def compute(slot, c):
        gb, db, _, _ = slot

        def group_body(g, carry):
            resv = jnp.zeros((L,), jnp.float32)
            for p16 in range(L):
                p = g * L + p16
                acc = jnp.zeros((L,), jnp.float32)
                for q in range(D // 32):
                    gv = plsc.bitcast(gb[p, pl.ds(L * q, L)], jnp.bfloat16)
                    dv = plsc.bitcast(db[p, pl.ds(L * q, L)], jnp.bfloat16)
                    ge, go = plsc.unpack(gv, format=plsc.PackFormat.INTERLEAVED,
                                         preferred_element_type=jnp.float32)
                    de, do = plsc.unpack(dv, format=plsc.PackFormat.INTERLEAVED,
                                         preferred_element_type=jnp.float32)
                    te = jnp.maximum(ge + de, 0.0)
                    acc = acc + te * w2r[2 * q]
                    to = jnp.maximum(go + do, 0.0)
                    acc = acc + to * w2r[2 * q + 1]
                s = jnp.sum(acc)
                resv = jnp.where(lane == p16, s, resv)
            outv[pl.ds(c * CH + g * L, L)] = resv + b2vec
            return carry

        lax.fori_loop(0, CH // L, group_body, 0)

    issue(slots[0], 0)
    issue(slots[1], 1)

    def pair_body(d, carry):
        c = 2 * d
        wait_slot(slots[0])
        compute(slots[0], c)

        @pl.when(c + 2 < NCH)
        def _():
            issue(slots[0], c + 2)

        wait_slot(slots[1])
        compute(slots[1], c + 1)

        @pl.when(c + 3 < NCH)
        def _():
            issue(slots[1], c + 3)

        return carry

    lax.fori_loop(0, NCH2, pair_body, 0)
    if NCH % 2:
        wait_slot(slots[0])
        compute(slots[0], NCH - 1)
    pltpu.sync_copy(outv, out_hbm.at[pl.ds(wid * BPW, BPW)])


def _pack_cols(t):
    a = jax.lax.bitcast_convert_type(
        t[:, :D // 2].astype(jnp.bfloat16), jnp.uint16).astype(jnp.int32)
    b = jax.lax.bitcast_convert_type(
        t[:, D // 2:].astype(jnp.bfloat16), jnp.uint16).astype(jnp.int32)
    return a | (b << 16)


def _tc_tables_body(xg_ref, wng_ref, bng_ref, mg_ref, xd_ref, wnd_ref, bnd_ref,
                    md_ref, w1a_ref, w1b_ref, b1_ref, rows_ref, cols_ref,
                    gp_ref, dp_ref):
    del rows_ref, cols_ref
    f32 = jnp.float32
    xn_g = jnp.dot(xg_ref[...], wng_ref[...], preferred_element_type=f32) + bng_ref[...]
    zg = jnp.maximum(xn_g, 0.0) * mg_ref[:, 0:1]
    gp_ref[...] = _pack_cols(jnp.dot(zg, w1a_ref[...], preferred_element_type=f32))
    xn_d = jnp.dot(xd_ref[...], wnd_ref[...], preferred_element_type=f32) + bnd_ref[...]
    zd = jnp.maximum(xn_d, 0.0) * md_ref[:, 0:1]
    dp_ref[...] = _pack_cols(
        jnp.dot(zd, w1b_ref[...], preferred_element_type=f32) + b1_ref[...])


def _tc_z_body(xg_ref, wng_ref, bng_ref, mg_ref, xd_ref, wnd_ref, bnd_ref,
               md_ref, zg_ref, zd_ref):
    f32 = jnp.float32
    xn_g = jnp.dot(xg_ref[...], wng_ref[...], preferred_element_type=f32) + bng_ref[...]
    zg_ref[...] = jnp.maximum(xn_g, 0.0) * mg_ref[:, 0:1]
    xn_d = jnp.dot(xd_ref[...], wnd_ref[...], preferred_element_type=f32) + bnd_ref[...]
    zd_ref[...] = jnp.maximum(xn_d, 0.0) * md_ref[:, 0:1]


_TC_R = 2000

_full = lambda i: (0, 0)
_rows = lambda i: (i, 0)
_spec_x = pl.BlockSpec((_TC_R, D), _rows)
_spec_w = pl.BlockSpec((D, D), _full)
_spec_b = pl.BlockSpec((1, D), _full)
_spec_p = pl.BlockSpec((_TC_R, D // 2), _rows)
_spec_m = pl.BlockSpec((_TC_R, D), _rows)
_spec_any = pl.BlockSpec(memory_space=pl.ANY)


def _tc_tables_call(xg, wng, bng, mg, xd, wnd, bnd, md, w1a, w1b, b1,
                    rows3, cols3):
    return pl.pallas_call(
        _tc_tables_body,
        grid=(NN // _TC_R,),
        in_specs=[_spec_x, _spec_w, _spec_b, _spec_m,
                  _spec_x, _spec_w, _spec_b, _spec_m,
                  _spec_w, _spec_w, _spec_b, _spec_any, _spec_any],
        out_specs=[_spec_p, _spec_p],
        out_shape=[jax.ShapeDtypeStruct((NN, D // 2), jnp.int32),
                   jax.ShapeDtypeStruct((NN, D // 2), jnp.int32)],
    )(xg, wng, bng, mg, xd, wnd, bnd, md, w1a, w1b, b1, rows3, cols3)


def _tc_z_call(xg, wng, bng, mg, xd, wnd, bnd, md):
    return pl.pallas_call(
        _tc_z_body,
        grid=(NN // _TC_R,),
        in_specs=[_spec_x, _spec_w, _spec_b, _spec_m,
                  _spec_x, _spec_w, _spec_b, _spec_m],
        out_specs=[_spec_x, _spec_x],
        out_shape=[jax.ShapeDtypeStruct((NN, D), jnp.float32),
                   jax.ShapeDtypeStruct((NN, D), jnp.float32)],
    )(xg, wng, bng, mg, xd, wnd, bnd, md)


def kernel(x_gene, x_disease, edge_index_g2d, edge_index_d2g, edge_label_index,
           edge_type_emb_g2d, edge_type_emb_d2g,
           Wn_gene, bn_gene, Wn_disease, bn_disease,
           We_g2d, be_g2d, We_d2g, be_d2g,
           att_w, att_b, W1, b1, W2, b2):
    i32 = jnp.int32
    eg = edge_index_g2d[1].astype(i32)
    ed = edge_index_d2g[1].astype(i32)

    row = edge_label_index[0].astype(i32)
    col = edge_label_index[1].astype(i32)
    pad = jnp.zeros((BPAD - NB,), i32)
    rows3 = jnp.concatenate([row, pad]).reshape(NW, NCH, CH)
    cols3 = jnp.concatenate([col, pad]).reshape(NW, NCH, CH)

    maskd_bc, maskg_bc = _edge_mask_kernel(eg, ed)

    gp, dp = _tc_tables_call(
        x_gene, Wn_gene, bn_gene.reshape(1, D), maskg_bc,
        x_disease, Wn_disease, bn_disease.reshape(1, D), maskd_bc,
        W1[:D], W1[D:], b1.reshape(1, D), rows3, cols3)
    zg, zd = _tc_z_call(
        x_gene, Wn_gene, bn_gene.reshape(1, D), maskg_bc,
        x_disease, Wn_disease, bn_disease.reshape(1, D), maskd_bc)
    w2flat = W2.reshape(2, D // 32, L).transpose(1, 0, 2).reshape(D)
    b2vec = jnp.broadcast_to(b2, (L,))

    predp = _decoder_kernel(gp, dp, rows3, cols3, w2flat, b2vec)
    return (predp[:NB], zg, zd)


# speedup vs baseline: 1.1119x; 1.1119x over previous
import functools

import jax
import jax.numpy as jnp
from jax import lax
from jax.experimental import pallas as pl
from jax.experimental.pallas import tpu as pltpu
from jax.experimental.pallas import tpu_sc as plsc

D = 128
NN = 10000
NNP = 10240
NRED = NNP // 16
NE = 320000
NB = 100000
NC, NS, L = 2, 16, 16
NW = NC * NS
BPW = 3200
BPAD = NW * BPW
CH = 64
NCH = BPW // CH
NCH2 = NCH // 2
EPT = NE // NS
RJ = D // L
MW = 16

_mesh = plsc.VectorSubcoreMesh(core_axis_name="c", subcore_axis_name="s")
_sc_params = pltpu.CompilerParams(needs_layout_passes=False)
_sc_params_untiled = pltpu.CompilerParams(needs_layout_passes=False,
                                          use_tc_tiling_on_sc=False)


@functools.partial(
    pl.kernel,
    out_type=(jax.ShapeDtypeStruct((NNP, D), jnp.float32),
              jax.ShapeDtypeStruct((NNP, D), jnp.float32)),
    mesh=_mesh,
    scratch_types=[
        pltpu.VMEM((EPT,), jnp.int32),
        pltpu.VMEM((NNP,), jnp.float32),
        pltpu.VMEM((NS, NRED), jnp.float32),
        pltpu.VMEM((NRED,), jnp.float32),
        pltpu.VMEM((128, D), jnp.float32),
        pltpu.VMEM_SHARED((NS, NNP), jnp.float32),
    ],
    compiler_params=_sc_params,
)
def _edge_mask_kernel(eidx_a, eidx_b, out_a, out_b, idx_v,
                      mask_v, red_v, res_v, bc_v, shared):
    cid = lax.axis_index("c")
    sid = lax.axis_index("s")
    zeros16 = jnp.zeros((L,), jnp.float32)
    ones16 = jnp.ones((L,), jnp.float32)

    def run(eidx_hbm, out_hbm):
        pltpu.sync_copy(eidx_hbm.at[1, sid], idx_v)

        def zero_body(i, carry):
            for u in range(4):
                mask_v[pl.ds((4 * i + u) * L, L)] = zeros16
            return carry

        lax.fori_loop(0, NNP // (4 * L), zero_body, 0)

        def scat_body(i, carry):
            for u in range(4):
                idx = idx_v[pl.ds((4 * i + u) * L, L)]
                plsc.store_scatter(mask_v, [idx], ones16)
            return carry

        lax.fori_loop(0, EPT // (4 * L), scat_body, 0)

        pltpu.sync_copy(mask_v, shared.at[sid])
        plsc.subcore_barrier()
        for s in range(NS):
            pltpu.sync_copy(shared.at[s, pl.ds(sid * NRED, NRED)], red_v.at[s])

        def red_body(j, carry):
            m = red_v[0, pl.ds(j * L, L)]
            for s in range(1, NS):
                m = jnp.maximum(m, red_v[s, pl.ds(j * L, L)])
            res_v[pl.ds(j * L, L)] = m
            return carry

        lax.fori_loop(0, NRED // L, red_body, 0)

        for ch in range(NRED // 128):
            def grp_body(k, carry):
                mv = res_v[pl.ds(ch * 128 + k * L, L)]
                for j in range(L):
                    bc_v[k * L + j, pl.ds(0, L)] = jnp.broadcast_to(mv[j], (L,))
                return carry

            lax.fori_loop(0, 128 // L, grp_body, 0)
            pltpu.sync_copy(bc_v, out_hbm.at[pl.ds(sid * NRED + ch * 128, 128)])

    @pl.when(cid == 0)
    def _():
        run(eidx_a, out_a)

    @pl.when(cid == 1)
    def _():
        run(eidx_b, out_b)


@functools.partial(
    pl.kernel,
    out_type=jax.ShapeDtypeStruct((BPAD,), jnp.float32),
    mesh=_mesh,
    scratch_types=[
        pltpu.VMEM((NCH, CH), jnp.int32),
        pltpu.VMEM((NCH, CH), jnp.int32),
        pltpu.VMEM((CH, D // 2), jnp.int32),
        pltpu.VMEM((CH, D // 2), jnp.int32),
        pltpu.VMEM((CH, D // 2), jnp.int32),
        pltpu.VMEM((CH, D // 2), jnp.int32),
        pltpu.VMEM((BPW,), jnp.float32),
        pltpu.VMEM((D,), jnp.float32),
        pltpu.VMEM((L,), jnp.float32),
        pltpu.SemaphoreType.DMA,
        pltpu.SemaphoreType.DMA,
        pltpu.SemaphoreType.DMA,
        pltpu.SemaphoreType.DMA,
    ],
    compiler_params=_sc_params_untiled,
)
def _decoder_kernel(g_hbm, dz_hbm, rows_hbm, cols_hbm, w2_hbm, b2_hbm, out_hbm,
                    rowi_v, coli_v, gbuf0, dbuf0, gbuf1, dbuf1, outv, w2_v,
                    b2_v, sg0, sd0, sg1, sd1):
    wid = lax.axis_index("s") * NC + lax.axis_index("c")
    pltpu.sync_copy(rows_hbm.at[wid], rowi_v)
    pltpu.sync_copy(cols_hbm.at[wid], coli_v)
    pltpu.sync_copy(w2_hbm, w2_v)
    pltpu.sync_copy(b2_hbm, b2_v)
    b2vec = b2_v[...]
    w2r = [w2_v[pl.ds(L * j, L)] for j in range(RJ)]
    lane = lax.iota(jnp.int32, L)
    slots = ((gbuf0, dbuf0, sg0, sd0), (gbuf1, dbuf1, sg1, sd1))

    def issue(slot, c):
        gb, db, sg, sd = slot
        pltpu.async_copy(g_hbm.at[rowi_v.at[c]], gb, sg)
        pltpu.async_copy(dz_hbm.at[coli_v.at[c]], db, sd)

    def wait_slot(slot):
        gb, db, sg, sd = slot
        pltpu.make_async_copy(g_hbm.at[rowi_v.at[0]], gb, sg).wait()
        pltpu.make_async_copy(dz_hbm.at[coli_v.at[0]], db, sd).wait()

    def compute(slot, c):
        gb, db, _, _ = slot

        def group_body(g, carry):
            resv = jnp.zeros((L,), jnp.float32)
            for p16 in range(L):
                p = g * L + p16
                acc = jnp.zeros((L,), jnp.float32)
                for q in range(D // 32):
                    gv = plsc.bitcast(gb[p, pl.ds(L * q, L)], jnp.bfloat16)
                    dv = plsc.bitcast(db[p, pl.ds(L * q, L)], jnp.bfloat16)
                    ge, go = plsc.unpack(gv, format=plsc.PackFormat.INTERLEAVED,
                                         preferred_element_type=jnp.float32)
                    de, do = plsc.unpack(dv, format=plsc.PackFormat.INTERLEAVED,
                                         preferred_element_type=jnp.float32)
                    te = jnp.maximum(ge + de, 0.0)
                    acc = acc + te * w2r[2 * q]
                    to = jnp.maximum(go + do, 0.0)
                    acc = acc + to * w2r[2 * q + 1]
                s = jnp.sum(acc)
                resv = jnp.where(lane == p16, s, resv)
            outv[pl.ds(c * CH + g * L, L)] = resv + b2vec
            return carry

        lax.fori_loop(0, CH // L, group_body, 0)

    issue(slots[0], 0)
    issue(slots[1], 1)

    def pair_body(d, carry):
        c = 2 * d
        wait_slot(slots[0])
        compute(slots[0], c)

        @pl.when(c + 2 < NCH)
        def _():
            issue(slots[0], c + 2)

        wait_slot(slots[1])
        compute(slots[1], c + 1)

        @pl.when(c + 3 < NCH)
        def _():
            issue(slots[1], c + 3)

        return carry

    lax.fori_loop(0, NCH2, pair_body, 0)
    if NCH % 2:
        wait_slot(slots[0])
        compute(slots[0], NCH - 1)
    pltpu.sync_copy(outv, out_hbm.at[pl.ds(wid * BPW, BPW)])


def _pack_cols(t):
    a = jax.lax.bitcast_convert_type(
        t[:, :D // 2].astype(jnp.bfloat16), jnp.uint16).astype(jnp.int32)
    b = jax.lax.bitcast_convert_type(
        t[:, D // 2:].astype(jnp.bfloat16), jnp.uint16).astype(jnp.int32)
    return a | (b << 16)


def _tc_body(xg_ref, wng_ref, bng_ref, mg_ref, xd_ref, wnd_ref, bnd_ref,
             md_ref, w1a_ref, w1b_ref, b1_ref,
             zg_ref, zd_ref, gp_ref, dp_ref):
    f32 = jnp.float32
    xn_g = jnp.dot(xg_ref[...], wng_ref[...], preferred_element_type=f32) + bng_ref[...]
    zg = jnp.maximum(xn_g, 0.0) * mg_ref[:, 0:1]
    zg_ref[...] = zg
    gp_ref[...] = _pack_cols(jnp.dot(zg, w1a_ref[...], preferred_element_type=f32))
    xn_d = jnp.dot(xd_ref[...], wnd_ref[...], preferred_element_type=f32) + bnd_ref[...]
    zd = jnp.maximum(xn_d, 0.0) * md_ref[:, 0:1]
    zd_ref[...] = zd
    dp_ref[...] = _pack_cols(
        jnp.dot(zd, w1b_ref[...], preferred_element_type=f32) + b1_ref[...])


_TC_R = 2000

_full = lambda i: (0, 0)
_rows = lambda i: (i, 0)
_spec_x = pl.BlockSpec((_TC_R, D), _rows)
_spec_w = pl.BlockSpec((D, D), _full)
_spec_b = pl.BlockSpec((1, D), _full)
_spec_p = pl.BlockSpec((_TC_R, D // 2), _rows)
_spec_m = pl.BlockSpec((_TC_R, D), _rows)


def _tc_call(xg, wng, bng, mg, xd, wnd, bnd, md, w1a, w1b, b1):
    return pl.pallas_call(
        _tc_body,
        grid=(NN // _TC_R,),
        in_specs=[_spec_x, _spec_w, _spec_b, _spec_m,
                  _spec_x, _spec_w, _spec_b, _spec_m,
                  _spec_w, _spec_w, _spec_b],
        out_specs=[_spec_x, _spec_x, _spec_p, _spec_p],
        out_shape=[jax.ShapeDtypeStruct((NN, D), jnp.float32),
                   jax.ShapeDtypeStruct((NN, D), jnp.float32),
                   jax.ShapeDtypeStruct((NN, D // 2), jnp.int32),
                   jax.ShapeDtypeStruct((NN, D // 2), jnp.int32)],
    )(xg, wng, bng, mg, xd, wnd, bnd, md, w1a, w1b, b1)


def kernel(x_gene, x_disease, edge_index_g2d, edge_index_d2g, edge_label_index,
           edge_type_emb_g2d, edge_type_emb_d2g,
           Wn_gene, bn_gene, Wn_disease, bn_disease,
           We_g2d, be_g2d, We_d2g, be_d2g,
           att_w, att_b, W1, b1, W2, b2):
    i32 = jnp.int32
    eg = edge_index_g2d.astype(i32).reshape(2, NS, EPT)
    ed = edge_index_d2g.astype(i32).reshape(2, NS, EPT)

    row = edge_label_index[0].astype(i32)
    col = edge_label_index[1].astype(i32)
    pad = jnp.zeros((BPAD - NB,), i32)
    rows3 = jnp.concatenate([row, pad]).reshape(NW, NCH, CH)
    cols3 = jnp.concatenate([col, pad]).reshape(NW, NCH, CH)

    maskd_bc, maskg_bc = _edge_mask_kernel(eg, ed)

    zg, zd, gp, dp = _tc_call(
        x_gene, Wn_gene, bn_gene.reshape(1, D), maskg_bc,
        x_disease, Wn_disease, bn_disease.reshape(1, D), maskd_bc,
        W1[:D], W1[D:], b1.reshape(1, D))
    w2flat = W2.reshape(2, D // 32, L).transpose(1, 0, 2).reshape(D)
    b2vec = jnp.broadcast_to(b2, (L,))

    predp = _decoder_kernel(gp, dp, rows3, cols3, w2flat, b2vec)
    return (predp[:NB], zg, zd)


# speedup vs baseline: 1.1446x; 1.0294x over previous
import functools

import jax
import jax.numpy as jnp
from jax import lax
from jax.experimental import pallas as pl
from jax.experimental.pallas import tpu as pltpu
from jax.experimental.pallas import tpu_sc as plsc

D = 128
NN = 10000
NNP = 10240
NRED = NNP // 16
NE = 320000
NB = 100000
NC, NS, L = 2, 16, 16
NW = NC * NS
BPW = 3200
BPAD = NW * BPW
CH = 32
NCH = BPW // CH
NCH2 = NCH // 2
EPT = NE // NS
RJ = D // L
MW = 16

_mesh = plsc.VectorSubcoreMesh(core_axis_name="c", subcore_axis_name="s")
_sc_params = pltpu.CompilerParams(needs_layout_passes=False)
_sc_params_untiled = pltpu.CompilerParams(needs_layout_passes=False,
                                          use_tc_tiling_on_sc=False)


@functools.partial(
    pl.kernel,
    out_type=(jax.ShapeDtypeStruct((NNP, D), jnp.float32),
              jax.ShapeDtypeStruct((NNP, D), jnp.float32)),
    mesh=_mesh,
    scratch_types=[
        pltpu.VMEM((EPT,), jnp.int32),
        pltpu.VMEM((NNP,), jnp.float32),
        pltpu.VMEM((NS, NRED), jnp.float32),
        pltpu.VMEM((NRED,), jnp.float32),
        pltpu.VMEM((128, D), jnp.float32),
        pltpu.VMEM_SHARED((NS, NNP), jnp.float32),
    ],
    compiler_params=_sc_params,
)
def _edge_mask_kernel(eidx_a, eidx_b, out_a, out_b, idx_v,
                      mask_v, red_v, res_v, bc_v, shared):
    cid = lax.axis_index("c")
    sid = lax.axis_index("s")
    zeros16 = jnp.zeros((L,), jnp.float32)
    ones16 = jnp.ones((L,), jnp.float32)

    def run(eidx_hbm, out_hbm):
        pltpu.sync_copy(eidx_hbm.at[1, sid], idx_v)

        def zero_body(i, carry):
            for u in range(4):
                mask_v[pl.ds((4 * i + u) * L, L)] = zeros16
            return carry

        lax.fori_loop(0, NNP // (4 * L), zero_body, 0)

        def scat_body(i, carry):
            for u in range(4):
                idx = idx_v[pl.ds((4 * i + u) * L, L)]
                plsc.store_scatter(mask_v, [idx], ones16)
            return carry

        lax.fori_loop(0, EPT // (4 * L), scat_body, 0)

        pltpu.sync_copy(mask_v, shared.at[sid])
        plsc.subcore_barrier()
        for s in range(NS):
            pltpu.sync_copy(shared.at[s, pl.ds(sid * NRED, NRED)], red_v.at[s])

        def red_body(j, carry):
            m = red_v[0, pl.ds(j * L, L)]
            for s in range(1, NS):
                m = jnp.maximum(m, red_v[s, pl.ds(j * L, L)])
            res_v[pl.ds(j * L, L)] = m
            return carry

        lax.fori_loop(0, NRED // L, red_body, 0)

        for ch in range(NRED // 128):
            def grp_body(k, carry):
                mv = res_v[pl.ds(ch * 128 + k * L, L)]
                for j in range(L):
                    bc_v[k * L + j, pl.ds(0, L)] = jnp.broadcast_to(mv[j], (L,))
                return carry

            lax.fori_loop(0, 128 // L, grp_body, 0)
            pltpu.sync_copy(bc_v, out_hbm.at[pl.ds(sid * NRED + ch * 128, 128)])

    @pl.when(cid == 0)
    def _():
        run(eidx_a, out_a)

    @pl.when(cid == 1)
    def _():
        run(eidx_b, out_b)


@functools.partial(
    pl.kernel,
    out_type=jax.ShapeDtypeStruct((BPAD,), jnp.float32),
    mesh=_mesh,
    scratch_types=[
        pltpu.VMEM((NCH, CH), jnp.int32),
        pltpu.VMEM((NCH, CH), jnp.int32),
        pltpu.VMEM((CH, D // 2), jnp.int32),
        pltpu.VMEM((CH, D // 2), jnp.int32),
        pltpu.VMEM((CH, D // 2), jnp.int32),
        pltpu.VMEM((CH, D // 2), jnp.int32),
        pltpu.VMEM((BPW,), jnp.float32),
        pltpu.VMEM((D,), jnp.float32),
        pltpu.VMEM((L,), jnp.float32),
        pltpu.SemaphoreType.DMA,
        pltpu.SemaphoreType.DMA,
        pltpu.SemaphoreType.DMA,
        pltpu.SemaphoreType.DMA,
    ],
    compiler_params=_sc_params_untiled,
)
def _decoder_kernel(g_hbm, dz_hbm, rows_hbm, cols_hbm, w2_hbm, b2_hbm, out_hbm,
                    rowi_v, coli_v, gbuf0, dbuf0, gbuf1, dbuf1, outv, w2_v,
                    b2_v, sg0, sd0, sg1, sd1):
    wid = lax.axis_index("s") * NC + lax.axis_index("c")
    pltpu.sync_copy(rows_hbm.at[wid], rowi_v)
    pltpu.sync_copy(cols_hbm.at[wid], coli_v)
    pltpu.sync_copy(w2_hbm, w2_v)
    pltpu.sync_copy(b2_hbm, b2_v)
    b2vec = b2_v[...]
    w2r = [w2_v[pl.ds(L * j, L)] for j in range(RJ)]
    lane = lax.iota(jnp.int32, L)
    slots = ((gbuf0, dbuf0, sg0, sd0), (gbuf1, dbuf1, sg1, sd1))

    def issue(slot, c):
        gb, db, sg, sd = slot
        pltpu.async_copy(g_hbm.at[rowi_v.at[c]], gb, sg)
        pltpu.async_copy(dz_hbm.at[coli_v.at[c]], db, sd)

    def wait_slot(slot):
        gb, db, sg, sd = slot
        pltpu.make_async_copy(g_hbm.at[rowi_v.at[0]], gb, sg).wait()
        pltpu.make_async_copy(dz_hbm.at[coli_v.at[0]], db, sd).wait()

    def compute(slot, c):
        gb, db, _, _ = slot

        def group_body(g, carry):
            resv = jnp.zeros((L,), jnp.float32)
            for p16 in range(L):
                p = g * L + p16
                acc = jnp.zeros((L,), jnp.float32)
                for q in range(D // 32):
                    gv = plsc.bitcast(gb[p, pl.ds(L * q, L)], jnp.bfloat16)
                    dv = plsc.bitcast(db[p, pl.ds(L * q, L)], jnp.bfloat16)
                    ge, go = plsc.unpack(gv, format=plsc.PackFormat.INTERLEAVED,
                                         preferred_element_type=jnp.float32)
                    de, do = plsc.unpack(dv, format=plsc.PackFormat.INTERLEAVED,
                                         preferred_element_type=jnp.float32)
                    te = jnp.maximum(ge + de, 0.0)
                    acc = acc + te * w2r[2 * q]
                    to = jnp.maximum(go + do, 0.0)
                    acc = acc + to * w2r[2 * q + 1]
                s = jnp.sum(acc)
                resv = jnp.where(lane == p16, s, resv)
            outv[pl.ds(c * CH + g * L, L)] = resv + b2vec
            return carry

        lax.fori_loop(0, CH // L, group_body, 0)

    issue(slots[0], 0)
    issue(slots[1], 1)

    def pair_body(d, carry):
        c = 2 * d
        wait_slot(slots[0])
        compute(slots[0], c)

        @pl.when(c + 2 < NCH)
        def _():
            issue(slots[0], c + 2)

        wait_slot(slots[1])
        compute(slots[1], c + 1)

        @pl.when(c + 3 < NCH)
        def _():
            issue(slots[1], c + 3)

        return carry

    lax.fori_loop(0, NCH2, pair_body, 0)
    if NCH % 2:
        wait_slot(slots[0])
        compute(slots[0], NCH - 1)
    pltpu.sync_copy(outv, out_hbm.at[pl.ds(wid * BPW, BPW)])


def _pack_cols(t):
    a = jax.lax.bitcast_convert_type(
        t[:, :D // 2].astype(jnp.bfloat16), jnp.uint16).astype(jnp.int32)
    b = jax.lax.bitcast_convert_type(
        t[:, D // 2:].astype(jnp.bfloat16), jnp.uint16).astype(jnp.int32)
    return a | (b << 16)


def _tc_body(xg_ref, wng_ref, bng_ref, mg_ref, xd_ref, wnd_ref, bnd_ref,
             md_ref, w1a_ref, w1b_ref, b1_ref,
             zg_ref, zd_ref, gp_ref, dp_ref):
    f32 = jnp.float32
    xn_g = jnp.dot(xg_ref[...], wng_ref[...], preferred_element_type=f32) + bng_ref[...]
    zg = jnp.maximum(xn_g, 0.0) * mg_ref[:, 0:1]
    zg_ref[...] = zg
    gp_ref[...] = _pack_cols(jnp.dot(zg, w1a_ref[...], preferred_element_type=f32))
    xn_d = jnp.dot(xd_ref[...], wnd_ref[...], preferred_element_type=f32) + bnd_ref[...]
    zd = jnp.maximum(xn_d, 0.0) * md_ref[:, 0:1]
    zd_ref[...] = zd
    dp_ref[...] = _pack_cols(
        jnp.dot(zd, w1b_ref[...], preferred_element_type=f32) + b1_ref[...])


_TC_R = 2000

_full = lambda i: (0, 0)
_rows = lambda i: (i, 0)
_spec_x = pl.BlockSpec((_TC_R, D), _rows)
_spec_w = pl.BlockSpec((D, D), _full)
_spec_b = pl.BlockSpec((1, D), _full)
_spec_p = pl.BlockSpec((_TC_R, D // 2), _rows)
_spec_m = pl.BlockSpec((_TC_R, D), _rows)


def _tc_call(xg, wng, bng, mg, xd, wnd, bnd, md, w1a, w1b, b1):
    return pl.pallas_call(
        _tc_body,
        grid=(NN // _TC_R,),
        in_specs=[_spec_x, _spec_w, _spec_b, _spec_m,
                  _spec_x, _spec_w, _spec_b, _spec_m,
                  _spec_w, _spec_w, _spec_b],
        out_specs=[_spec_x, _spec_x, _spec_p, _spec_p],
        out_shape=[jax.ShapeDtypeStruct((NN, D), jnp.float32),
                   jax.ShapeDtypeStruct((NN, D), jnp.float32),
                   jax.ShapeDtypeStruct((NN, D // 2), jnp.int32),
                   jax.ShapeDtypeStruct((NN, D // 2), jnp.int32)],
    )(xg, wng, bng, mg, xd, wnd, bnd, md, w1a, w1b, b1)


def kernel(x_gene, x_disease, edge_index_g2d, edge_index_d2g, edge_label_index,
           edge_type_emb_g2d, edge_type_emb_d2g,
           Wn_gene, bn_gene, Wn_disease, bn_disease,
           We_g2d, be_g2d, We_d2g, be_d2g,
           att_w, att_b, W1, b1, W2, b2):
    i32 = jnp.int32
    eg = edge_index_g2d.astype(i32).reshape(2, NS, EPT)
    ed = edge_index_d2g.astype(i32).reshape(2, NS, EPT)

    row = edge_label_index[0].astype(i32)
    col = edge_label_index[1].astype(i32)
    pad = jnp.zeros((BPAD - NB,), i32)
    rows3 = jnp.concatenate([row, pad]).reshape(NW, NCH, CH)
    cols3 = jnp.concatenate([col, pad]).reshape(NW, NCH, CH)

    maskd_bc, maskg_bc = _edge_mask_kernel(eg, ed)

    zg, zd, gp, dp = _tc_call(
        x_gene, Wn_gene, bn_gene.reshape(1, D), maskg_bc,
        x_disease, Wn_disease, bn_disease.reshape(1, D), maskd_bc,
        W1[:D], W1[D:], b1.reshape(1, D))
    w2flat = W2.reshape(2, D // 32, L).transpose(1, 0, 2).reshape(D)
    b2vec = jnp.broadcast_to(b2, (L,))

    predp = _decoder_kernel(gp, dp, rows3, cols3, w2flat, b2vec)
    return (predp[:NB], zg, zd)
